# baseline (speedup 1.0000x reference)
import jax
import jax.numpy as jnp
from jax.experimental import pallas as pl

_ROWS_PER_BLOCK = 16
_N_ITERS = 8


def _entmax15_block(x_ref, o_ref):
    xs = x_ref[...] * 0.5
    m = jnp.max(xs, axis=-1, keepdims=True)
    tau0 = m - 1.0

    def body(_, tau):
        mask = xs > tau
        v = jnp.where(mask, xs, 0.0)
        k = jnp.sum(mask.astype(jnp.float32), axis=-1, keepdims=True)
        s1 = jnp.sum(v, axis=-1, keepdims=True)
        s2 = jnp.sum(v * v, axis=-1, keepdims=True)
        disc = jnp.maximum(s1 * s1 - k * (s2 - 1.0), 0.0)
        k_safe = jnp.maximum(k, 1.0)
        tau_new = (s1 - jnp.sqrt(disc)) / k_safe
        return jnp.clip(tau_new, m - 1.0, m)

    tau = jax.lax.fori_loop(0, _N_ITERS, body, tau0)
    r = jnp.maximum(xs - tau, 0.0)
    o_ref[...] = r * r


def kernel(logits):
    b, n = logits.shape
    return pl.pallas_call(
        _entmax15_block,
        grid=(b // _ROWS_PER_BLOCK,),
        in_specs=[pl.BlockSpec((_ROWS_PER_BLOCK, n), lambda i: (i, 0))],
        out_specs=pl.BlockSpec((_ROWS_PER_BLOCK, n), lambda i: (i, 0)),
        out_shape=jax.ShapeDtypeStruct((b, n), logits.dtype),
    )(logits)


# speedup vs baseline: 1.0897x; 1.0897x over previous
import jax
import jax.numpy as jnp
from jax.experimental import pallas as pl

_ROWS_PER_BLOCK = 32
_N_ITERS = 8


def _entmax15_block(x_ref, o_ref):
    xs = x_ref[...] * 0.5
    m = jnp.max(xs, axis=-1, keepdims=True)
    tau0 = m - 1.0

    def body(_, tau):
        mask = xs > tau
        v = jnp.where(mask, xs, 0.0)
        k = jnp.sum(mask.astype(jnp.float32), axis=-1, keepdims=True)
        s1 = jnp.sum(v, axis=-1, keepdims=True)
        s2 = jnp.sum(v * v, axis=-1, keepdims=True)
        disc = jnp.maximum(s1 * s1 - k * (s2 - 1.0), 0.0)
        k_safe = jnp.maximum(k, 1.0)
        tau_new = (s1 - jnp.sqrt(disc)) / k_safe
        return jnp.clip(tau_new, m - 1.0, m)

    tau = jax.lax.fori_loop(0, _N_ITERS, body, tau0)
    r = jnp.maximum(xs - tau, 0.0)
    o_ref[...] = r * r


def kernel(logits):
    b, n = logits.shape
    return pl.pallas_call(
        _entmax15_block,
        grid=(b // _ROWS_PER_BLOCK,),
        in_specs=[pl.BlockSpec((_ROWS_PER_BLOCK, n), lambda i: (i, 0))],
        out_specs=pl.BlockSpec((_ROWS_PER_BLOCK, n), lambda i: (i, 0)),
        out_shape=jax.ShapeDtypeStruct((b, n), logits.dtype),
    )(logits)


# speedup vs baseline: 1.0981x; 1.0077x over previous
import jax
import jax.numpy as jnp
from jax.experimental import pallas as pl

_ROWS_PER_BLOCK = 64
_N_ITERS = 8


def _entmax15_block(x_ref, o_ref):
    xs = x_ref[...] * 0.5
    m = jnp.max(xs, axis=-1, keepdims=True)
    tau0 = m - 1.0

    def body(_, tau):
        mask = xs > tau
        v = jnp.where(mask, xs, 0.0)
        k = jnp.sum(mask.astype(jnp.float32), axis=-1, keepdims=True)
        s1 = jnp.sum(v, axis=-1, keepdims=True)
        s2 = jnp.sum(v * v, axis=-1, keepdims=True)
        disc = jnp.maximum(s1 * s1 - k * (s2 - 1.0), 0.0)
        k_safe = jnp.maximum(k, 1.0)
        tau_new = (s1 - jnp.sqrt(disc)) / k_safe
        return jnp.clip(tau_new, m - 1.0, m)

    tau = jax.lax.fori_loop(0, _N_ITERS, body, tau0)
    r = jnp.maximum(xs - tau, 0.0)
    o_ref[...] = r * r


def kernel(logits):
    b, n = logits.shape
    return pl.pallas_call(
        _entmax15_block,
        grid=(b // _ROWS_PER_BLOCK,),
        in_specs=[pl.BlockSpec((_ROWS_PER_BLOCK, n), lambda i: (i, 0))],
        out_specs=pl.BlockSpec((_ROWS_PER_BLOCK, n), lambda i: (i, 0)),
        out_shape=jax.ShapeDtypeStruct((b, n), logits.dtype),
    )(logits)


# speedup vs baseline: 1.4156x; 1.2891x over previous
import jax
import jax.numpy as jnp
from jax.experimental import pallas as pl

_ROWS_PER_BLOCK = 64
_N_NEWTON = 8


def _entmax15_block(x_ref, o_ref):
    x = x_ref[...]
    m = jnp.max(x, axis=-1, keepdims=True)
    u0 = m - 2.0

    def body(_, u):
        r = jnp.maximum(x - u, 0.0)
        f = jnp.sum(r * r, axis=-1, keepdims=True) - 4.0
        g = jnp.sum(r, axis=-1, keepdims=True) * 2.0
        un = u + f / jnp.maximum(g, 1e-30)
        return jnp.clip(un, m - 2.0, m)

    u = jax.lax.fori_loop(0, _N_NEWTON, body, u0)
    r = jnp.maximum(x - u, 0.0) * 0.5
    o_ref[...] = r * r


def kernel(logits):
    b, n = logits.shape
    return pl.pallas_call(
        _entmax15_block,
        grid=(b // _ROWS_PER_BLOCK,),
        in_specs=[pl.BlockSpec((_ROWS_PER_BLOCK, n), lambda i: (i, 0))],
        out_specs=pl.BlockSpec((_ROWS_PER_BLOCK, n), lambda i: (i, 0)),
        out_shape=jax.ShapeDtypeStruct((b, n), logits.dtype),
    )(logits)
